# baseline (speedup 1.0000x reference)
import functools

import jax
import jax.numpy as jnp
from jax import lax
from jax.experimental import pallas as pl
from jax.experimental.pallas import tpu as pltpu
from jax.experimental.pallas import tpu_sc as plsc

B = 10000
S = 10
D = 128
N_EMBED = 128
NLANE = 16
NREG = D // NLANE

NC = 2
NS = 16
NW = NC * NS

B_PAD = 10240
NODES_PER_W = B_PAD // NW

CELL = 120
NPC = CELL // S
NBUF = 5

SELF_CHUNKS = 3
SELF_SIZES = (120, 120, 80)
SELF_OFFS = (0, 120, 240)

NEIGH_CHUNKS = 27
LAST_NODES = NODES_PER_W - (NEIGH_CHUNKS - 1) * NPC
TOTAL_CHUNKS = SELF_CHUNKS + NEIGH_CHUNKS


def _sc_fused_body(self_idx_hbm, neigh_idx_hbm, stab_hbm, ntab_hbm, alpha_hbm,
                   self_out_hbm, agg_out_hbm,
                   idx_s_v, idx_n_v, rows_v, agg_v, asf_v, alpha_v, scr_v,
                   sem_g, sem_o):
    w = lax.axis_index("s") * NC + lax.axis_index("c")
    pltpu.sync_copy(self_idx_hbm.at[w], idx_s_v)
    pltpu.sync_copy(neigh_idx_hbm.at[w], idx_n_v)
    pltpu.sync_copy(alpha_hbm, alpha_v)

    self_base = w * NODES_PER_W

    iota = lax.iota(jnp.int32, 16)
    lane_mask = iota < S
    a1 = [alpha_v[pl.ds(NLANE * j, NLANE)] for j in range(NREG)]
    a2 = [alpha_v[pl.ds(D + NLANE * j, NLANE)] for j in range(NREG)]

    def buf(c, n=CELL):
        return rows_v.at[pl.ds(lax.rem(c, NBUF) * CELL, n)]

    def fire_gather(c):
        if isinstance(c, int) and c < SELF_CHUNKS:
            pltpu.async_copy(stab_hbm.at[idx_s_v.at[c]], buf(c),
                             sem_g.at[c % NBUF])
        else:
            pltpu.async_copy(ntab_hbm.at[idx_n_v.at[c - SELF_CHUNKS]], buf(c),
                             sem_g.at[lax.rem(c, NBUF)])

    def wait_gather(c):
        pltpu.make_async_copy(agg_out_hbm.at[pl.ds(0, CELL)], buf(c),
                              sem_g.at[lax.rem(c, NBUF)]).wait()

    def fire_out_self(c):
        dst = self_out_hbm.at[pl.ds(self_base + SELF_OFFS[c], SELF_SIZES[c])]
        pltpu.async_copy(buf(c, SELF_SIZES[c]), dst, sem_o.at[c % NBUF])

    def wait_out_self(c):
        dst = self_out_hbm.at[pl.ds(self_base, SELF_SIZES[c])]
        pltpu.make_async_copy(buf(c, SELF_SIZES[c]), dst,
                              sem_o.at[c % NBUF]).wait()

    def row_partial(row, avecs):
        p = avecs[0] * rows_v[row, pl.ds(0, NLANE)]
        for j in range(1, NREG):
            p = p + avecs[j] * rows_v[row, pl.ds(NLANE * j, NLANE)]
        return p

    def lane_sum(p, slot):
        cur = p
        for r, sh in enumerate((8, 4, 2)):
            base = slot * 96 + r * 32
            scr_v[pl.ds(base, NLANE)] = cur
            cur = cur + scr_v[pl.ds(base + sh, NLANE)]
        return cur[0] + cur[1]

    def compute_aself(c):
        base = (c % NBUF) * CELL
        off = SELF_OFFS[c]

        @pl.loop(0, SELF_SIZES[c])
        def _node(i):
            asf_v[off + i] = lane_sum(row_partial(base + i, a1), 0)

    def compute_neigh(c, n_nodes):
        base = lax.rem(c, NBUF) * CELL
        node0 = (c - SELF_CHUNKS) * NPC

        @pl.loop(0, n_nodes)
        def _node(i):
            row0 = base + i * S
            apos = node0 + i
            a_s = asf_v[apos]
            lv = jnp.zeros((NLANE,), jnp.float32)
            for s in range(S):
                ls = lane_sum(row_partial(row0 + s, a2), s) + a_s
                lv = jnp.where(iota == s, ls, lv)
            wv = jnp.where(lane_mask, jnp.exp(jnp.maximum(lv, 0.0)), 0.0)
            wn = wv / lane_sum(wv, S)
            wbs = [wn[s] for s in range(S)]
            for j in range(NREG):
                ds_j = pl.ds(NLANE * j, NLANE)
                acc = wbs[0] * rows_v[row0, ds_j]
                for s in range(1, S):
                    acc = acc + wbs[s] * rows_v[row0 + s, ds_j]
                agg_v[apos, ds_j] = acc

    for c in range(NBUF):
        fire_gather(c)
    for c in range(SELF_CHUNKS):
        wait_gather(c)
        fire_out_self(c)
        compute_aself(c)
    wait_out_self(0); fire_gather(NBUF)
    wait_out_self(1); fire_gather(NBUF + 1)
    wait_out_self(2); fire_gather(NBUF + 2)
    wait_gather(SELF_CHUNKS)
    compute_neigh(SELF_CHUNKS, NPC)

    @pl.loop(SELF_CHUNKS + 1, TOTAL_CHUNKS - NBUF + 1)
    def _steady(c):
        fire_gather(c + NBUF - 1)
        wait_gather(c)
        compute_neigh(c, NPC)

    for c in range(TOTAL_CHUNKS - NBUF + 1, TOTAL_CHUNKS):
        wait_gather(c)
        compute_neigh(c, NPC if c < TOTAL_CHUNKS - 1 else LAST_NODES)

    pltpu.sync_copy(agg_v, agg_out_hbm.at[pl.ds(self_base, NODES_PER_W)])


@functools.cache
def _sc_fused():
    return pl.kernel(
        _sc_fused_body,
        out_type=(
            jax.ShapeDtypeStruct((B_PAD, D), jnp.float32),
            jax.ShapeDtypeStruct((B_PAD, D), jnp.float32),
        ),
        mesh=plsc.VectorSubcoreMesh(
            core_axis_name="c", subcore_axis_name="s",
            num_cores=NC, num_subcores=NS),
        scratch_types=[
            pltpu.VMEM((SELF_CHUNKS, CELL), jnp.int32),
            pltpu.VMEM((NEIGH_CHUNKS, CELL), jnp.int32),
            pltpu.VMEM((NBUF * CELL, D), jnp.float32),
            pltpu.VMEM((NODES_PER_W, D), jnp.float32),
            pltpu.SMEM((NODES_PER_W,), jnp.float32),
            pltpu.VMEM((2 * D,), jnp.float32),
            pltpu.VMEM(((S + 1) * 96,), jnp.float32),
            pltpu.SemaphoreType.DMA((NBUF,)),
            pltpu.SemaphoreType.DMA((NBUF,)),
        ],
    )


BLK = 256
GRID = B_PAD // BLK


def _tc_linear_body(self_ref, agg_ref, w1t_ref, w2t_ref, out_ref):
    out = (jnp.dot(self_ref[...], w1t_ref[...],
                   preferred_element_type=jnp.float32)
           + jnp.dot(agg_ref[...], w2t_ref[...],
                     preferred_element_type=jnp.float32))
    out_ref[...] = jnp.maximum(out, 0.0)


@jax.jit
def kernel(nodes, neigh_index, self_feat_table, neigh_feat_table, weight,
           alpha):
    nodes_pad = jnp.zeros((B_PAD,), jnp.int32).at[:B].set(nodes)
    self_idx = jnp.zeros((NW, SELF_CHUNKS * CELL), jnp.int32)
    self_idx = self_idx.at[:, :NODES_PER_W].set(
        nodes_pad.reshape(NW, NODES_PER_W))
    self_idx = self_idx.reshape(NW, SELF_CHUNKS, CELL)

    ni_pad = jnp.zeros((B_PAD, S), jnp.int32).at[:B].set(neigh_index)
    neigh_rows_per_w = NODES_PER_W * S
    neigh_idx = jnp.zeros((NW, NEIGH_CHUNKS * CELL), jnp.int32)
    neigh_idx = neigh_idx.at[:, :neigh_rows_per_w].set(
        ni_pad.reshape(NW, neigh_rows_per_w))
    neigh_idx = neigh_idx.reshape(NW, NEIGH_CHUNKS, CELL)

    x, agg = _sc_fused()(
        self_idx, neigh_idx, self_feat_table, neigh_feat_table,
        alpha.reshape(2 * D))

    w1t = weight[:, :D].T
    w2t = weight[:, D:].T
    out = pl.pallas_call(
        _tc_linear_body,
        out_shape=jax.ShapeDtypeStruct((B, N_EMBED), jnp.float32),
        grid=(GRID,),
        in_specs=[
            pl.BlockSpec((BLK, D), lambda i: (i, 0)),
            pl.BlockSpec((BLK, D), lambda i: (i, 0)),
            pl.BlockSpec((D, N_EMBED), lambda i: (0, 0)),
            pl.BlockSpec((D, N_EMBED), lambda i: (0, 0)),
        ],
        out_specs=pl.BlockSpec((BLK, N_EMBED), lambda i: (i, 0)),
    )(x, agg, w1t, w2t)

    return out
